# baseline (speedup 1.0000x reference)
import functools

import jax
import jax.numpy as jnp
from jax import lax
from jax.experimental import pallas as pl
from jax.experimental.pallas import tpu as pltpu
from jax.experimental.pallas import tpu_sc as plsc

N = 10000
D = 256
H = 128
HP = 128
G = 64
E = 160000
VOCAB = 10000
N_AST = 6000
N_STMT = 4000

NP = 10240
RXA = NP // 16
NACC = 10112
STRIPE = NACC // 16
EC = 128
ECH = 79
EPT = EC * ECH
EPAD = 16 * EPT

RB = 1000
NBLK = N // RB
SB = 1000
SBLK = N_STMT // SB

_NEG = -1e30


def _dot(a, b):
    return jax.lax.dot_general(a, b, (((1,), (0,)), ((), ())),
                               preferred_element_type=jnp.float32)


def _dot_t(a, b):
    return jax.lax.dot_general(a, b, (((0,), (0,)), ((), ())),
                               preferred_element_type=jnp.float32)


def _stmt_body(sf_ref, w_ref, b_ref, out_ref):
    out_ref[...] = jnp.tanh(_dot(sf_ref[...], w_ref[...]) + b_ref[...])


def _stmt_encode(stmt_feats, w, b):
    return pl.pallas_call(
        _stmt_body,
        grid=(SBLK,),
        in_specs=[
            pl.BlockSpec((SB, D), lambda i: (i, 0)),
            pl.BlockSpec((D, D), lambda i: (0, 0)),
            pl.BlockSpec((1, D), lambda i: (0, 0)),
        ],
        out_specs=pl.BlockSpec((SB, D), lambda i: (i, 0)),
        out_shape=jax.ShapeDtypeStruct((N_STMT, D), jnp.float32),
        compiler_params=pltpu.CompilerParams(
            dimension_semantics=("arbitrary",)),
    )(stmt_feats, w, b.reshape(1, D))


_SC_MESH = plsc.VectorSubcoreMesh(core_axis_name="c", subcore_axis_name="s")


def _sparse_body(t2_hbm, g2_hbm, src2_hbm, dst3_hbm, zz_hbm,
                 x_out_hbm, s_out_hbm,
                 gbuf, srcbuf, dstbuf, rows, acc, sem):
    c = lax.axis_index("c")
    s = lax.axis_index("s")

    pltpu.sync_copy(zz_hbm, acc.at[pl.ds(s * STRIPE, STRIPE)])
    pltpu.sync_copy(src2_hbm.at[c, pl.ds(s * EPT, EPT)], srcbuf)
    pltpu.sync_copy(dst3_hbm.at[s], dstbuf)

    pltpu.sync_copy(g2_hbm.at[c, pl.ds(s * RXA, RXA)], gbuf)
    xbase = c * NP + s * RXA
    for k in range(RXA // EC):
        pltpu.async_copy(t2_hbm.at[gbuf.at[pl.ds(k * EC, EC)]],
                         rows, sem).wait()
        pltpu.sync_copy(rows, x_out_hbm.at[pl.ds(xbase + k * EC, EC)])

    plsc.subcore_barrier()

    def chunk_step(j, _):
        idx = srcbuf.at[pl.ds(j * EC, EC)]
        pltpu.async_copy(x_out_hbm.at[idx], rows, sem).wait()
        pltpu.sync_copy(rows, acc.at[dstbuf.at[j]], add=True)
        return 0

    lax.fori_loop(0, ECH, chunk_step, 0)

    plsc.subcore_barrier()

    pltpu.sync_copy(acc.at[pl.ds(s * STRIPE, STRIPE)],
                    s_out_hbm.at[c, pl.ds(s * STRIPE, STRIPE)])


def _sparse_phase(t2, g2, src2, dst3, zz):
    return pl.kernel(
        _sparse_body,
        out_type=[
            jax.ShapeDtypeStruct((2 * NP, HP), jnp.float32),
            jax.ShapeDtypeStruct((2, NACC, HP), jnp.float32),
        ],
        mesh=_SC_MESH,
        scratch_types=[
            pltpu.VMEM((RXA,), jnp.int32),
            pltpu.VMEM((EPT,), jnp.int32),
            pltpu.VMEM((ECH, EC), jnp.int32),
            pltpu.VMEM((EC, HP), jnp.float32),
            pltpu.VMEM_SHARED((NACC, HP), jnp.float32),
            pltpu.SemaphoreType.DMA,
        ],
    )(t2, g2, src2, dst3, zz)


def _h_body(xl_ref, xr_ref, sl_ref, sr_ref, deg_ref, lst_ref, wself_ref,
            wmsg_ref, h_ref, q_ref, qacc):
    i = pl.program_id(0)

    @pl.when(i == 0)
    def _():
        qacc[...] = jnp.zeros_like(qacc)

    xl = xl_ref[0]
    xr = xr_ref[0]
    sl = sl_ref[0]
    sr = sr_ref[0]
    inv = 1.0 / jnp.clip(deg_ref[...], 1.0, None)
    ws = wself_ref[...]
    wm = wmsg_ref[...]
    agg = (_dot(sl, wm[:H, :]) + _dot(sr, wm[H:, :])) * inv
    h = jax.nn.relu(_dot(xl, ws[:H, :]) + _dot(xr, ws[H:, :]) + agg)
    h_ref[...] = h

    rows = jax.lax.broadcasted_iota(jnp.int32, (RB, G), 0) + i * RB
    onehot = (rows == lst_ref[...]).astype(jnp.float32)
    qacc[...] += _dot_t(onehot, h)

    @pl.when(i == NBLK - 1)
    def _():
        q_ref[...] = qacc[...]


def _h_and_q(x_stack, s_stack, deg_col, last_stmts, w_self, w_msg):
    return pl.pallas_call(
        _h_body,
        grid=(NBLK,),
        in_specs=[
            pl.BlockSpec((1, RB, HP), lambda i: (0, i, 0)),
            pl.BlockSpec((1, RB, HP), lambda i: (1, i, 0)),
            pl.BlockSpec((1, RB, HP), lambda i: (0, i, 0)),
            pl.BlockSpec((1, RB, HP), lambda i: (1, i, 0)),
            pl.BlockSpec((RB, 1), lambda i: (i, 0)),
            pl.BlockSpec((1, G), lambda i: (0, 0)),
            pl.BlockSpec((D, D), lambda i: (0, 0)),
            pl.BlockSpec((D, D), lambda i: (0, 0)),
        ],
        out_specs=[
            pl.BlockSpec((RB, D), lambda i: (i, 0)),
            pl.BlockSpec((G, D), lambda i: (0, 0)),
        ],
        out_shape=[
            jax.ShapeDtypeStruct((N, D), jnp.float32),
            jax.ShapeDtypeStruct((G, D), jnp.float32),
        ],
        scratch_shapes=[pltpu.VMEM((G, D), jnp.float32)],
        compiler_params=pltpu.CompilerParams(
            dimension_semantics=("arbitrary",)),
    )(x_stack, x_stack, s_stack, s_stack, deg_col,
      last_stmts.reshape(1, G).astype(jnp.int32), w_self, w_msg)


def _pool_body(h_ref, q_ref, gid_ref, wproj_ref, bproj_ref, wlab_ref,
               blab_ref, y_ref, m_sc, den_sc, num_sc):
    i = pl.program_id(0)

    @pl.when(i == 0)
    def _():
        m_sc[...] = jnp.full_like(m_sc, _NEG)
        den_sc[...] = jnp.zeros_like(den_sc)
        num_sc[...] = jnp.zeros_like(num_sc)

    h = h_ref[...]
    cols = jax.lax.broadcasted_iota(jnp.int32, (RB, G), 1)
    onehot = (cols == gid_ref[...]).astype(jnp.float32)
    qexp = _dot(onehot, q_ref[...])
    s = jnp.sum(h * qexp, axis=1, keepdims=True) / 16.0
    s_masked = jnp.where(onehot > 0.0, s, _NEG)
    blk_max = jnp.max(s_masked, axis=0, keepdims=True)
    m_old = m_sc[...]
    m_new = jnp.maximum(m_old, blk_max)
    scale = jnp.exp(m_old - m_new)
    p = jnp.exp(jnp.where(onehot > 0.0, s - m_new, _NEG))
    den_sc[...] = den_sc[...] * scale + jnp.sum(p, axis=0, keepdims=True)
    num_sc[...] = num_sc[...] * scale + _dot_t(h, p)
    m_sc[...] = m_new

    @pl.when(i == NBLK - 1)
    def _():
        zt = _dot_t(wproj_ref[...], num_sc[...] / den_sc[...]) + bproj_ref[...]
        dist = jnp.abs(zt[:, :G // 2] - zt[:, G // 2:])
        y = _dot_t(wlab_ref[...], dist) + blab_ref[...]
        y_ref[...] = jax.nn.sigmoid(y)


def _pool_head(h, q, gid_col, w_proj, b_proj, w_label, b_label):
    return pl.pallas_call(
        _pool_body,
        grid=(NBLK,),
        in_specs=[
            pl.BlockSpec((RB, D), lambda i: (i, 0)),
            pl.BlockSpec((G, D), lambda i: (0, 0)),
            pl.BlockSpec((RB, 1), lambda i: (i, 0)),
            pl.BlockSpec((D, D), lambda i: (0, 0)),
            pl.BlockSpec((D, 1), lambda i: (0, 0)),
            pl.BlockSpec((D, 1), lambda i: (0, 0)),
            pl.BlockSpec((1, 1), lambda i: (0, 0)),
        ],
        out_specs=pl.BlockSpec((1, G // 2), lambda i: (0, 0)),
        out_shape=jax.ShapeDtypeStruct((1, G // 2), jnp.float32),
        scratch_shapes=[
            pltpu.VMEM((1, G), jnp.float32),
            pltpu.VMEM((1, G), jnp.float32),
            pltpu.VMEM((D, G), jnp.float32),
        ],
        compiler_params=pltpu.CompilerParams(
            dimension_semantics=("arbitrary",)),
    )(h, q, gid_col, w_proj, b_proj.reshape(D, 1), w_label,
      b_label.reshape(1, 1))


def kernel(ast_node_index, batch_tree_index, stmt_feats, edge_index,
           graph_ids, last_stmts, ast_table, W_tbcnn, b_tbcnn, W_self,
           W_msg, W_proj, b_proj, w_label, b_label):
    ast_node_index = ast_node_index.astype(jnp.int32)
    order = jnp.argsort(jnp.concatenate(
        [ast_node_index, batch_tree_index.astype(jnp.int32)]))
    g = jnp.where(order < N_AST,
                  ast_node_index[jnp.clip(order, 0, N_AST - 1)],
                  VOCAB + order - N_AST)
    g_p = jnp.concatenate([g, jnp.zeros((NP - N,), jnp.int32)])

    stmt_emb = _stmt_encode(stmt_feats, W_tbcnn, b_tbcnn)

    t2 = jnp.concatenate([
        ast_table[:, :H], stmt_emb[:, :H],
        ast_table[:, H:], stmt_emb[:, H:],
    ], axis=0)

    g2 = jnp.stack([g_p, g_p + (VOCAB + N_STMT)])

    src = edge_index[0].astype(jnp.int32)
    dst = edge_index[1].astype(jnp.int32)
    src_p = jnp.concatenate([src, jnp.zeros((EPAD - E,), jnp.int32)])
    src2 = jnp.stack([src_p, src_p + NP])
    dst_p = jnp.concatenate([dst, jnp.full((EPAD - E,), N, jnp.int32)])
    dst3 = dst_p.reshape(16, ECH, EC)
    zz = jnp.zeros((STRIPE, HP), jnp.float32)

    x2, s_stack = _sparse_phase(t2, g2, src2, dst3, zz)
    x_stack = x2.reshape(2, NP, HP)
    deg = jax.ops.segment_sum(jnp.ones((E,), jnp.float32), dst, num_segments=N)

    h, q = _h_and_q(x_stack, s_stack, deg.reshape(N, 1), last_stmts,
                    W_self, W_msg)
    y = _pool_head(h, q, graph_ids.reshape(N, 1).astype(jnp.int32),
                   W_proj, b_proj, w_label, b_label)
    return y.reshape(G // 2)


# speedup vs baseline: 1.0115x; 1.0115x over previous
import functools

import jax
import jax.numpy as jnp
from jax import lax
from jax.experimental import pallas as pl
from jax.experimental.pallas import tpu as pltpu
from jax.experimental.pallas import tpu_sc as plsc

N = 10000
D = 256
H = 128
HP = 128
G = 64
E = 160000
VOCAB = 10000
N_AST = 6000
N_STMT = 4000

NP = 10240
RXA = NP // 16
NACC = 10112
STRIPE = NACC // 16
EC = 128
ECH = 79
EPT = EC * ECH
EPAD = 16 * EPT

RB = 1000
NBLK = N // RB
SB = 1000
SBLK = N_STMT // SB

_NEG = -1e30


def _dot(a, b):
    return jax.lax.dot_general(a, b, (((1,), (0,)), ((), ())),
                               preferred_element_type=jnp.float32)


def _dot_t(a, b):
    return jax.lax.dot_general(a, b, (((0,), (0,)), ((), ())),
                               preferred_element_type=jnp.float32)


def _stmt_body(sf_ref, w_ref, b_ref, out_ref):
    out_ref[...] = jnp.tanh(_dot(sf_ref[...], w_ref[...]) + b_ref[...])


def _stmt_encode(stmt_feats, w, b):
    return pl.pallas_call(
        _stmt_body,
        grid=(SBLK,),
        in_specs=[
            pl.BlockSpec((SB, D), lambda i: (i, 0)),
            pl.BlockSpec((D, D), lambda i: (0, 0)),
            pl.BlockSpec((1, D), lambda i: (0, 0)),
        ],
        out_specs=pl.BlockSpec((SB, D), lambda i: (i, 0)),
        out_shape=jax.ShapeDtypeStruct((N_STMT, D), jnp.float32),
        compiler_params=pltpu.CompilerParams(
            dimension_semantics=("arbitrary",)),
    )(stmt_feats, w, b.reshape(1, D))


_SC_MESH = plsc.VectorSubcoreMesh(core_axis_name="c", subcore_axis_name="s")


def _sparse_body(t2_hbm, order_hbm, ast_hbm, src2_hbm, dst3_hbm, zz_hbm,
                 x_out_hbm, s_out_hbm,
                 gbuf, abuf, srcbuf, dstbuf, rows, acc, sem):
    c = lax.axis_index("c")
    s = lax.axis_index("s")

    pltpu.sync_copy(zz_hbm, acc.at[pl.ds(s * STRIPE, STRIPE)])
    pltpu.sync_copy(src2_hbm.at[c, pl.ds(s * EPT, EPT)], srcbuf)
    pltpu.sync_copy(dst3_hbm.at[s], dstbuf)

    pltpu.sync_copy(order_hbm.at[pl.ds(s * RXA, RXA)], gbuf)
    pltpu.sync_copy(ast_hbm, abuf)
    coff = c * (VOCAB + N_STMT)

    def g_step(k, _):
        o = gbuf[pl.ds(k * 16, 16)]
        m = o < N_AST
        a = plsc.load_gather(abuf, [jnp.where(m, o, 0)])
        gbuf[pl.ds(k * 16, 16)] = jnp.where(m, a, VOCAB + o - N_AST) + coff
        return 0

    lax.fori_loop(0, RXA // 16, g_step, 0)

    xbase = c * NP + s * RXA
    for k in range(RXA // EC):
        pltpu.async_copy(t2_hbm.at[gbuf.at[pl.ds(k * EC, EC)]],
                         rows, sem).wait()
        pltpu.sync_copy(rows, x_out_hbm.at[pl.ds(xbase + k * EC, EC)])

    plsc.subcore_barrier()

    def chunk_step(j, _):
        idx = srcbuf.at[pl.ds(j * EC, EC)]
        pltpu.async_copy(x_out_hbm.at[idx], rows, sem).wait()
        pltpu.sync_copy(rows, acc.at[dstbuf.at[j]], add=True)
        return 0

    lax.fori_loop(0, ECH, chunk_step, 0)

    plsc.subcore_barrier()

    pltpu.sync_copy(acc.at[pl.ds(s * STRIPE, STRIPE)],
                    s_out_hbm.at[c, pl.ds(s * STRIPE, STRIPE)])


def _sparse_phase(t2, order_p, ast_idx, src2, dst3, zz):
    return pl.kernel(
        _sparse_body,
        out_type=[
            jax.ShapeDtypeStruct((2 * NP, HP), jnp.float32),
            jax.ShapeDtypeStruct((2, NACC, HP), jnp.float32),
        ],
        mesh=_SC_MESH,
        scratch_types=[
            pltpu.VMEM((RXA,), jnp.int32),
            pltpu.VMEM((N_AST,), jnp.int32),
            pltpu.VMEM((EPT,), jnp.int32),
            pltpu.VMEM((ECH, EC), jnp.int32),
            pltpu.VMEM((EC, HP), jnp.float32),
            pltpu.VMEM_SHARED((NACC, HP), jnp.float32),
            pltpu.SemaphoreType.DMA,
        ],
        compiler_params=pltpu.CompilerParams(needs_layout_passes=False),
    )(t2, order_p, ast_idx, src2, dst3, zz)


def _h_body(xl_ref, xr_ref, sl_ref, sr_ref, deg_ref, lst_ref, wself_ref,
            wmsg_ref, h_ref, q_ref, qacc):
    i = pl.program_id(0)

    @pl.when(i == 0)
    def _():
        qacc[...] = jnp.zeros_like(qacc)

    xl = xl_ref[0]
    xr = xr_ref[0]
    sl = sl_ref[0]
    sr = sr_ref[0]
    inv = 1.0 / jnp.clip(deg_ref[...], 1.0, None)
    ws = wself_ref[...]
    wm = wmsg_ref[...]
    agg = (_dot(sl, wm[:H, :]) + _dot(sr, wm[H:, :])) * inv
    h = jax.nn.relu(_dot(xl, ws[:H, :]) + _dot(xr, ws[H:, :]) + agg)
    h_ref[...] = h

    rows = jax.lax.broadcasted_iota(jnp.int32, (RB, G), 0) + i * RB
    onehot = (rows == lst_ref[...]).astype(jnp.float32)
    qacc[...] += _dot_t(onehot, h)

    @pl.when(i == NBLK - 1)
    def _():
        q_ref[...] = qacc[...]


def _h_and_q(x_stack, s_stack, deg_col, last_stmts, w_self, w_msg):
    return pl.pallas_call(
        _h_body,
        grid=(NBLK,),
        in_specs=[
            pl.BlockSpec((1, RB, HP), lambda i: (0, i, 0)),
            pl.BlockSpec((1, RB, HP), lambda i: (1, i, 0)),
            pl.BlockSpec((1, RB, HP), lambda i: (0, i, 0)),
            pl.BlockSpec((1, RB, HP), lambda i: (1, i, 0)),
            pl.BlockSpec((RB, 1), lambda i: (i, 0)),
            pl.BlockSpec((1, G), lambda i: (0, 0)),
            pl.BlockSpec((D, D), lambda i: (0, 0)),
            pl.BlockSpec((D, D), lambda i: (0, 0)),
        ],
        out_specs=[
            pl.BlockSpec((RB, D), lambda i: (i, 0)),
            pl.BlockSpec((G, D), lambda i: (0, 0)),
        ],
        out_shape=[
            jax.ShapeDtypeStruct((N, D), jnp.float32),
            jax.ShapeDtypeStruct((G, D), jnp.float32),
        ],
        scratch_shapes=[pltpu.VMEM((G, D), jnp.float32)],
        compiler_params=pltpu.CompilerParams(
            dimension_semantics=("arbitrary",)),
    )(x_stack, x_stack, s_stack, s_stack, deg_col,
      last_stmts.reshape(1, G).astype(jnp.int32), w_self, w_msg)


def _pool_body(h_ref, q_ref, gid_ref, wproj_ref, bproj_ref, wlab_ref,
               blab_ref, y_ref, m_sc, den_sc, num_sc):
    i = pl.program_id(0)

    @pl.when(i == 0)
    def _():
        m_sc[...] = jnp.full_like(m_sc, _NEG)
        den_sc[...] = jnp.zeros_like(den_sc)
        num_sc[...] = jnp.zeros_like(num_sc)

    h = h_ref[...]
    cols = jax.lax.broadcasted_iota(jnp.int32, (RB, G), 1)
    onehot = (cols == gid_ref[...]).astype(jnp.float32)
    qexp = _dot(onehot, q_ref[...])
    s = jnp.sum(h * qexp, axis=1, keepdims=True) / 16.0
    s_masked = jnp.where(onehot > 0.0, s, _NEG)
    blk_max = jnp.max(s_masked, axis=0, keepdims=True)
    m_old = m_sc[...]
    m_new = jnp.maximum(m_old, blk_max)
    scale = jnp.exp(m_old - m_new)
    p = jnp.exp(jnp.where(onehot > 0.0, s - m_new, _NEG))
    den_sc[...] = den_sc[...] * scale + jnp.sum(p, axis=0, keepdims=True)
    num_sc[...] = num_sc[...] * scale + _dot_t(h, p)
    m_sc[...] = m_new

    @pl.when(i == NBLK - 1)
    def _():
        zt = _dot_t(wproj_ref[...], num_sc[...] / den_sc[...]) + bproj_ref[...]
        dist = jnp.abs(zt[:, :G // 2] - zt[:, G // 2:])
        y = _dot_t(wlab_ref[...], dist) + blab_ref[...]
        y_ref[...] = jax.nn.sigmoid(y)


def _pool_head(h, q, gid_col, w_proj, b_proj, w_label, b_label):
    return pl.pallas_call(
        _pool_body,
        grid=(NBLK,),
        in_specs=[
            pl.BlockSpec((RB, D), lambda i: (i, 0)),
            pl.BlockSpec((G, D), lambda i: (0, 0)),
            pl.BlockSpec((RB, 1), lambda i: (i, 0)),
            pl.BlockSpec((D, D), lambda i: (0, 0)),
            pl.BlockSpec((D, 1), lambda i: (0, 0)),
            pl.BlockSpec((D, 1), lambda i: (0, 0)),
            pl.BlockSpec((1, 1), lambda i: (0, 0)),
        ],
        out_specs=pl.BlockSpec((1, G // 2), lambda i: (0, 0)),
        out_shape=jax.ShapeDtypeStruct((1, G // 2), jnp.float32),
        scratch_shapes=[
            pltpu.VMEM((1, G), jnp.float32),
            pltpu.VMEM((1, G), jnp.float32),
            pltpu.VMEM((D, G), jnp.float32),
        ],
        compiler_params=pltpu.CompilerParams(
            dimension_semantics=("arbitrary",)),
    )(h, q, gid_col, w_proj, b_proj.reshape(D, 1), w_label,
      b_label.reshape(1, 1))


def kernel(ast_node_index, batch_tree_index, stmt_feats, edge_index,
           graph_ids, last_stmts, ast_table, W_tbcnn, b_tbcnn, W_self,
           W_msg, W_proj, b_proj, w_label, b_label):
    ast_node_index = ast_node_index.astype(jnp.int32)
    order = jnp.argsort(jnp.concatenate(
        [ast_node_index, batch_tree_index.astype(jnp.int32)]))
    order_p = jnp.concatenate([order, jnp.zeros((NP - N,), jnp.int32)])

    stmt_emb = _stmt_encode(stmt_feats, W_tbcnn, b_tbcnn)

    t2 = jnp.concatenate([
        ast_table[:, :H], stmt_emb[:, :H],
        ast_table[:, H:], stmt_emb[:, H:],
    ], axis=0)

    src = edge_index[0].astype(jnp.int32)
    dst = edge_index[1].astype(jnp.int32)
    src_p = jnp.concatenate([src, jnp.zeros((EPAD - E,), jnp.int32)])
    src2 = jnp.stack([src_p, src_p + NP])
    dst_p = jnp.concatenate([dst, jnp.full((EPAD - E,), N, jnp.int32)])
    dst3 = dst_p.reshape(16, ECH, EC)
    zz = jnp.zeros((STRIPE, HP), jnp.float32)

    x2, s_stack = _sparse_phase(t2, order_p, ast_node_index, src2, dst3, zz)
    x_stack = x2.reshape(2, NP, HP)
    deg = jax.ops.segment_sum(jnp.ones((E,), jnp.float32), dst, num_segments=N)

    h, q = _h_and_q(x_stack, s_stack, deg.reshape(N, 1), last_stmts,
                    W_self, W_msg)
    y = _pool_head(h, q, graph_ids.reshape(N, 1).astype(jnp.int32),
                   W_proj, b_proj, w_label, b_label)
    return y.reshape(G // 2)


# speedup vs baseline: 1.0567x; 1.0447x over previous
import functools

import jax
import jax.numpy as jnp
from jax import lax
from jax.experimental import pallas as pl
from jax.experimental.pallas import tpu as pltpu
from jax.experimental.pallas import tpu_sc as plsc

N = 10000
D = 256
H = 128
HP = 128
G = 64
E = 160000
VOCAB = 10000
N_AST = 6000
N_STMT = 4000

NP = 10240
RXA = NP // 16
NACC = 10112
STRIPE = NACC // 16
EC = 128
ECH = 79
EPT = EC * ECH
EPAD = 16 * EPT

RB = 1000
NBLK = N // RB
SB = 1000
SBLK = N_STMT // SB

_NEG = -1e30


def _dot(a, b):
    return jax.lax.dot_general(a, b, (((1,), (0,)), ((), ())),
                               preferred_element_type=jnp.float32)


def _dot_t(a, b):
    return jax.lax.dot_general(a, b, (((0,), (0,)), ((), ())),
                               preferred_element_type=jnp.float32)


def _stmt_body(sf_ref, w_ref, b_ref, out_ref):
    out_ref[...] = jnp.tanh(_dot(sf_ref[...], w_ref[...]) + b_ref[...])


def _stmt_encode(stmt_feats, w, b):
    return pl.pallas_call(
        _stmt_body,
        grid=(SBLK,),
        in_specs=[
            pl.BlockSpec((SB, D), lambda i: (i, 0)),
            pl.BlockSpec((D, D), lambda i: (0, 0)),
            pl.BlockSpec((1, D), lambda i: (0, 0)),
        ],
        out_specs=pl.BlockSpec((SB, D), lambda i: (i, 0)),
        out_shape=jax.ShapeDtypeStruct((N_STMT, D), jnp.float32),
        compiler_params=pltpu.CompilerParams(
            dimension_semantics=("arbitrary",)),
    )(stmt_feats, w, b.reshape(1, D))


_SC_MESH = plsc.VectorSubcoreMesh(core_axis_name="c", subcore_axis_name="s")

RA = NP // 32


def _xgather_body(t2_hbm, order_hbm, ast_hbm, x_out_hbm,
                  gbuf, gbuf2, abuf, rows_l, rows_r, sem):
    c = lax.axis_index("c")
    s = lax.axis_index("s")
    t = s * 2 + c
    base = t * RA

    pltpu.sync_copy(order_hbm.at[pl.ds(base, RA)], gbuf)
    pltpu.sync_copy(ast_hbm, abuf)

    def g_step(k, _):
        o = gbuf[pl.ds(k * 16, 16)]
        m = o < N_AST
        a = plsc.load_gather(abuf, [jnp.where(m, o, 0)])
        g = jnp.where(m, a, VOCAB + o - N_AST)
        gbuf[pl.ds(k * 16, 16)] = g
        gbuf2[pl.ds(k * 16, 16)] = g + (VOCAB + N_STMT)
        return 0

    lax.fori_loop(0, RA // 16, g_step, 0)

    for off, sz in ((0, 128), (128, 128), (256, 64)):
        pltpu.async_copy(t2_hbm.at[gbuf.at[pl.ds(off, sz)]],
                         rows_l.at[pl.ds(0, sz)], sem).wait()
        pltpu.sync_copy(rows_l.at[pl.ds(0, sz)],
                        x_out_hbm.at[pl.ds(base + off, sz)])
        pltpu.async_copy(t2_hbm.at[gbuf2.at[pl.ds(off, sz)]],
                         rows_r.at[pl.ds(0, sz)], sem).wait()
        pltpu.sync_copy(rows_r.at[pl.ds(0, sz)],
                        x_out_hbm.at[pl.ds(NP + base + off, sz)])


def _xgather(t2, order_p, ast_idx):
    return pl.kernel(
        _xgather_body,
        out_type=jax.ShapeDtypeStruct((2 * NP, HP), jnp.float32),
        mesh=_SC_MESH,
        scratch_types=[
            pltpu.VMEM((RA,), jnp.int32),
            pltpu.VMEM((RA,), jnp.int32),
            pltpu.VMEM((N_AST,), jnp.int32),
            pltpu.VMEM((128, HP), jnp.float32),
            pltpu.VMEM((128, HP), jnp.float32),
            pltpu.SemaphoreType.DMA,
        ],
        compiler_params=pltpu.CompilerParams(needs_layout_passes=False),
    )(t2, order_p, ast_idx)


def _edge_body(x2_hbm, src2_hbm, dst3_hbm, zz_hbm, s_out_hbm,
               srcbuf, dstbuf, rows, acc, sem):
    c = lax.axis_index("c")
    s = lax.axis_index("s")

    pltpu.sync_copy(zz_hbm, acc.at[pl.ds(s * STRIPE, STRIPE)])
    pltpu.sync_copy(src2_hbm.at[c, pl.ds(s * EPT, EPT)], srcbuf)
    pltpu.sync_copy(dst3_hbm.at[s], dstbuf)

    plsc.subcore_barrier()

    def chunk_step(j, _):
        idx = srcbuf.at[pl.ds(j * EC, EC)]
        pltpu.async_copy(x2_hbm.at[idx], rows, sem).wait()
        pltpu.sync_copy(rows, acc.at[dstbuf.at[j]], add=True)
        return 0

    lax.fori_loop(0, ECH, chunk_step, 0)

    plsc.subcore_barrier()

    pltpu.sync_copy(acc.at[pl.ds(s * STRIPE, STRIPE)],
                    s_out_hbm.at[c, pl.ds(s * STRIPE, STRIPE)])


def _edge_segsum(x2, src2, dst3, zz):
    return pl.kernel(
        _edge_body,
        out_type=jax.ShapeDtypeStruct((2, NACC, HP), jnp.float32),
        mesh=_SC_MESH,
        scratch_types=[
            pltpu.VMEM((EPT,), jnp.int32),
            pltpu.VMEM((ECH, EC), jnp.int32),
            pltpu.VMEM((EC, HP), jnp.float32),
            pltpu.VMEM_SHARED((NACC, HP), jnp.float32),
            pltpu.SemaphoreType.DMA,
        ],
    )(x2, src2, dst3, zz)


def _h_body(xl_ref, xr_ref, sl_ref, sr_ref, deg_ref, lst_ref, wself_ref,
            wmsg_ref, h_ref, q_ref, qacc):
    i = pl.program_id(0)

    @pl.when(i == 0)
    def _():
        qacc[...] = jnp.zeros_like(qacc)

    xl = xl_ref[0]
    xr = xr_ref[0]
    sl = sl_ref[0]
    sr = sr_ref[0]
    inv = 1.0 / jnp.clip(deg_ref[...], 1.0, None)
    ws = wself_ref[...]
    wm = wmsg_ref[...]
    agg = (_dot(sl, wm[:H, :]) + _dot(sr, wm[H:, :])) * inv
    h = jax.nn.relu(_dot(xl, ws[:H, :]) + _dot(xr, ws[H:, :]) + agg)
    h_ref[...] = h

    rows = jax.lax.broadcasted_iota(jnp.int32, (RB, G), 0) + i * RB
    onehot = (rows == lst_ref[...]).astype(jnp.float32)
    qacc[...] += _dot_t(onehot, h)

    @pl.when(i == NBLK - 1)
    def _():
        q_ref[...] = qacc[...]


def _h_and_q(x_stack, s_stack, deg_col, last_stmts, w_self, w_msg):
    return pl.pallas_call(
        _h_body,
        grid=(NBLK,),
        in_specs=[
            pl.BlockSpec((1, RB, HP), lambda i: (0, i, 0)),
            pl.BlockSpec((1, RB, HP), lambda i: (1, i, 0)),
            pl.BlockSpec((1, RB, HP), lambda i: (0, i, 0)),
            pl.BlockSpec((1, RB, HP), lambda i: (1, i, 0)),
            pl.BlockSpec((RB, 1), lambda i: (i, 0)),
            pl.BlockSpec((1, G), lambda i: (0, 0)),
            pl.BlockSpec((D, D), lambda i: (0, 0)),
            pl.BlockSpec((D, D), lambda i: (0, 0)),
        ],
        out_specs=[
            pl.BlockSpec((RB, D), lambda i: (i, 0)),
            pl.BlockSpec((G, D), lambda i: (0, 0)),
        ],
        out_shape=[
            jax.ShapeDtypeStruct((N, D), jnp.float32),
            jax.ShapeDtypeStruct((G, D), jnp.float32),
        ],
        scratch_shapes=[pltpu.VMEM((G, D), jnp.float32)],
        compiler_params=pltpu.CompilerParams(
            dimension_semantics=("arbitrary",)),
    )(x_stack, x_stack, s_stack, s_stack, deg_col,
      last_stmts.reshape(1, G).astype(jnp.int32), w_self, w_msg)


def _pool_body(h_ref, q_ref, gid_ref, wproj_ref, bproj_ref, wlab_ref,
               blab_ref, y_ref, m_sc, den_sc, num_sc):
    i = pl.program_id(0)

    @pl.when(i == 0)
    def _():
        m_sc[...] = jnp.full_like(m_sc, _NEG)
        den_sc[...] = jnp.zeros_like(den_sc)
        num_sc[...] = jnp.zeros_like(num_sc)

    h = h_ref[...]
    cols = jax.lax.broadcasted_iota(jnp.int32, (RB, G), 1)
    onehot = (cols == gid_ref[...]).astype(jnp.float32)
    qexp = _dot(onehot, q_ref[...])
    s = jnp.sum(h * qexp, axis=1, keepdims=True) / 16.0
    s_masked = jnp.where(onehot > 0.0, s, _NEG)
    blk_max = jnp.max(s_masked, axis=0, keepdims=True)
    m_old = m_sc[...]
    m_new = jnp.maximum(m_old, blk_max)
    scale = jnp.exp(m_old - m_new)
    p = jnp.exp(jnp.where(onehot > 0.0, s - m_new, _NEG))
    den_sc[...] = den_sc[...] * scale + jnp.sum(p, axis=0, keepdims=True)
    num_sc[...] = num_sc[...] * scale + _dot_t(h, p)
    m_sc[...] = m_new

    @pl.when(i == NBLK - 1)
    def _():
        zt = _dot_t(wproj_ref[...], num_sc[...] / den_sc[...]) + bproj_ref[...]
        dist = jnp.abs(zt[:, :G // 2] - zt[:, G // 2:])
        y = _dot_t(wlab_ref[...], dist) + blab_ref[...]
        y_ref[...] = jax.nn.sigmoid(y)


def _pool_head(h, q, gid_col, w_proj, b_proj, w_label, b_label):
    return pl.pallas_call(
        _pool_body,
        grid=(NBLK,),
        in_specs=[
            pl.BlockSpec((RB, D), lambda i: (i, 0)),
            pl.BlockSpec((G, D), lambda i: (0, 0)),
            pl.BlockSpec((RB, 1), lambda i: (i, 0)),
            pl.BlockSpec((D, D), lambda i: (0, 0)),
            pl.BlockSpec((D, 1), lambda i: (0, 0)),
            pl.BlockSpec((D, 1), lambda i: (0, 0)),
            pl.BlockSpec((1, 1), lambda i: (0, 0)),
        ],
        out_specs=pl.BlockSpec((1, G // 2), lambda i: (0, 0)),
        out_shape=jax.ShapeDtypeStruct((1, G // 2), jnp.float32),
        scratch_shapes=[
            pltpu.VMEM((1, G), jnp.float32),
            pltpu.VMEM((1, G), jnp.float32),
            pltpu.VMEM((D, G), jnp.float32),
        ],
        compiler_params=pltpu.CompilerParams(
            dimension_semantics=("arbitrary",)),
    )(h, q, gid_col, w_proj, b_proj.reshape(D, 1), w_label,
      b_label.reshape(1, 1))


def kernel(ast_node_index, batch_tree_index, stmt_feats, edge_index,
           graph_ids, last_stmts, ast_table, W_tbcnn, b_tbcnn, W_self,
           W_msg, W_proj, b_proj, w_label, b_label):
    ast_node_index = ast_node_index.astype(jnp.int32)
    order = jnp.argsort(jnp.concatenate(
        [ast_node_index, batch_tree_index.astype(jnp.int32)]))
    order_p = jnp.concatenate([order, jnp.zeros((NP - N,), jnp.int32)])

    stmt_emb = _stmt_encode(stmt_feats, W_tbcnn, b_tbcnn)

    t2 = jnp.concatenate([
        ast_table[:, :H], stmt_emb[:, :H],
        ast_table[:, H:], stmt_emb[:, H:],
    ], axis=0)

    src = edge_index[0].astype(jnp.int32)
    dst = edge_index[1].astype(jnp.int32)
    src_p = jnp.concatenate([src, jnp.zeros((EPAD - E,), jnp.int32)])
    src2 = jnp.stack([src_p, src_p + NP])
    dst_p = jnp.concatenate([dst, jnp.full((EPAD - E,), N, jnp.int32)])
    dst3 = dst_p.reshape(16, ECH, EC)
    zz = jnp.zeros((STRIPE, HP), jnp.float32)

    x2 = _xgather(t2, order_p, ast_node_index)
    x_stack = x2.reshape(2, NP, HP)
    s_stack = _edge_segsum(x2, src2, dst3, zz)
    deg = jax.ops.segment_sum(jnp.ones((E,), jnp.float32), dst, num_segments=N)

    h, q = _h_and_q(x_stack, s_stack, deg.reshape(N, 1), last_stmts,
                    W_self, W_msg)
    y = _pool_head(h, q, graph_ids.reshape(N, 1).astype(jnp.int32),
                   W_proj, b_proj, w_label, b_label)
    return y.reshape(G // 2)
